# Initial kernel scaffold; baseline (speedup 1.0000x reference)
import jax
import jax.numpy as jnp
from jax.experimental import pallas as pl


def kernel(inputs, targets):
    raise NotImplementedError("write your pallas kernel here")


# speedup vs baseline: 1.0817x; 1.0817x over previous
import jax
import jax.numpy as jnp
from jax.experimental import pallas as pl
from jax.experimental.pallas import tpu as pltpu

_G = 128
_RES = 0.25
_NG = 16
_N = _G * _G
_KT = 2048
_JT = 2048
_NKT = _N // _KT
_BIG = 3.0e38


def _chamfer_body(a_ref, b_ref, out_ref, colmin_ref):
    g = pl.program_id(0)
    kt = pl.program_id(1)

    @pl.when(kt == 0)
    def _init_colmin():
        colmin_ref[...] = jnp.full_like(colmin_ref[...], jnp.float32(_BIG))

    @pl.when(jnp.logical_and(g == 0, kt == 0))
    def _init_out():
        out_ref[0, 0] = jnp.float32(0.0)

    a = a_ref[0]
    rowmin = jnp.full((_KT, 1), _BIG, dtype=jnp.float32)
    for jc in range(_N // _JT):
        b = b_ref[0, :, jc * _JT:(jc + 1) * _JT]
        d = jax.lax.dot_general(
            a, b, (((1,), (0,)), ((), ())),
            precision=jax.lax.Precision.HIGHEST,
            preferred_element_type=jnp.float32)
        rowmin = jnp.minimum(rowmin, jnp.min(d, axis=1, keepdims=True))
        cm = jnp.min(d, axis=0, keepdims=True)
        colmin_ref[0:1, jc * _JT:(jc + 1) * _JT] = jnp.minimum(
            colmin_ref[0:1, jc * _JT:(jc + 1) * _JT], cm)

    out_ref[0, 0] += jnp.sum(rowmin)

    @pl.when(kt == _NKT - 1)
    def _flush_colmin():
        out_ref[0, 0] += jnp.sum(colmin_ref[...])

    @pl.when(jnp.logical_and(g == _NG - 1, kt == _NKT - 1))
    def _finalize():
        out_ref[0, 0] = out_ref[0, 0] * jnp.float32(1.0 / _NG)


def _chamfer_call(a, b):
    return pl.pallas_call(
        _chamfer_body,
        grid=(_NG, _NKT),
        in_specs=[
            pl.BlockSpec((1, _KT, 8), lambda g, kt: (g, kt, 0)),
            pl.BlockSpec((1, 8, _N), lambda g, kt: (g, 0, 0)),
        ],
        out_specs=pl.BlockSpec(
            (1, 1), lambda g, kt: (0, 0), memory_space=pltpu.SMEM),
        out_shape=jax.ShapeDtypeStruct((1, 1), jnp.float32),
        scratch_shapes=[pltpu.VMEM((1, _N), jnp.float32)],
        compiler_params=pltpu.CompilerParams(
            dimension_semantics=("arbitrary", "arbitrary")),
    )(a, b)


def kernel(inputs, targets):
    zp = inputs.reshape(_NG, _N).astype(jnp.float32)
    zg = targets.reshape(_NG, _N).astype(jnp.float32)

    k = jnp.arange(_N, dtype=jnp.int32)
    c = jnp.float32((_G - 1) * _RES * 0.5)
    gx = (k // _G).astype(jnp.float32) * _RES - c
    gy = (k % _G).astype(jnp.float32) * _RES - c
    xy2 = gx * gx + gy * gy

    ngt = xy2[None, :] + zg * zg
    npd = xy2[None, :] + zp * zp
    ones = jnp.ones((_NG, _N), jnp.float32)
    zero = jnp.zeros((_NG, _N), jnp.float32)
    bx = jnp.broadcast_to(gx[None, :], (_NG, _N))
    by = jnp.broadcast_to(gy[None, :], (_NG, _N))

    a = jnp.stack([-2.0 * bx, -2.0 * by, -2.0 * zg, ngt, ones,
                   zero, zero, zero], axis=-1)
    b = jnp.stack([bx, by, zp, ones, npd, zero, zero, zero],
                  axis=1)

    out = _chamfer_call(a, b)
    return out[0, 0]
